# baseline (speedup 1.0000x reference)
import functools

import jax
import jax.numpy as jnp
from jax import lax
from jax.experimental import pallas as pl
from jax.experimental.pallas import tpu as pltpu
from jax.experimental.pallas import tpu_sc as plsc

B, L, V, D = 16384, 200, 128, 8
N = B * L
NC, NS = 1, 16
NW = NC * NS
BPW = B // NW
NB = 2
CT = NB * L
NCHUNK = BPW // NB
NGF = L // 16
TAIL = L - NGF * 16
CTS = CT + 16


def _fuse_body(wte_ref, w_ref, b_ref, out_ref):
    out_ref[...] = lax.dot_general(
        wte_ref[...], w_ref[...],
        dimension_numbers=(((1,), (1,)), ((), ())),
        preferred_element_type=jnp.float32,
    ) + b_ref[...]


_fuse_table = pl.pallas_call(
    _fuse_body,
    out_shape=jax.ShapeDtypeStruct((V, D), jnp.float32),
)


def _sc_body(ft_hbm, idx_hbm, out_hbm, tbl_v, idx_v, out_v, sem_in, sem_out):
    wid = lax.axis_index("s") * NC + lax.axis_index("c")
    base = wid * BPW * L
    wb = wid * BPW

    pltpu.sync_copy(ft_hbm, tbl_v)

    pltpu.async_copy(
        idx_hbm.at[pl.ds(base, CT)], idx_v.at[pl.ds(0, CT)], sem_in)

    iota16 = lax.iota(jnp.int32, 16)
    dvecs = [jnp.full((16,), d, jnp.int32) for d in range(D)]
    tailmask = iota16 < TAIL

    @pl.loop(0, NCHUNK)
    def _chunk(c):
        slot = c % 2

        ioff = slot * CTS

        pltpu.make_async_copy(
            idx_hbm.at[pl.ds(base + c * CT, CT)],
            idx_v.at[pl.ds(ioff, CT)], sem_in).wait()

        @pl.when(c + 1 < NCHUNK)
        def _():
            pltpu.async_copy(
                idx_hbm.at[pl.ds(base + (c + 1) * CT, CT)],
                idx_v.at[pl.ds((1 - slot) * CTS, CT)], sem_in)

        @pl.when(c >= 2)
        def _():
            pltpu.make_async_copy(
                out_v.at[slot],
                out_hbm.at[pl.ds(wb + (c - 2) * NB, NB)], sem_out).wait()

        for jb in range(NB):
            row = out_v.at[slot, jb]

            @plsc.parallel_loop(0, NGF, unroll=6)
            def _grp(g):
                xv = idx_v[pl.ds(ioff + jb * L + g * 16, 16)]
                gbase = (xv & (V - 1)) * D
                lvec = iota16 + g * 16
                for d in range(D):
                    vals = plsc.load_gather(tbl_v, [gbase + d])
                    plsc.store_scatter(row, [lvec, dvecs[d]], vals)

            xv = idx_v[pl.ds(ioff + jb * L + NGF * 16, 16)]
            gbase = (xv & (V - 1)) * D
            lvec = iota16 + NGF * 16
            for d in range(D):
                vals = plsc.load_gather(tbl_v, [gbase + d])
                plsc.store_scatter(row, [lvec, dvecs[d]], vals, mask=tailmask)

        pltpu.async_copy(
            out_v.at[slot],
            out_hbm.at[pl.ds(wb + c * NB, NB)], sem_out)

    for t in (NCHUNK - 2, NCHUNK - 1):
        pltpu.make_async_copy(
            out_v.at[t % 2],
            out_hbm.at[pl.ds(wb + t * NB, NB)], sem_out).wait()


_sc_gather = pl.kernel(
    _sc_body,
    out_type=jax.ShapeDtypeStruct((B, L, D), jnp.float32),
    mesh=plsc.VectorSubcoreMesh(
        core_axis_name="c", subcore_axis_name="s",
        num_cores=NC, num_subcores=NS),
    compiler_params=pltpu.CompilerParams(needs_layout_passes=False),
    scratch_types=[
        pltpu.VMEM((V * D,), jnp.float32),
        pltpu.VMEM((2 * CTS,), jnp.int32),
        pltpu.VMEM((2, NB, L, D), jnp.float32),
        pltpu.SemaphoreType.DMA,
        pltpu.SemaphoreType.DMA,
    ],
)


@jax.jit
def kernel(x, wte, W, b):
    ft = _fuse_table(wte, W, b.reshape(1, D))
    return _sc_gather(ft.reshape(V * D), x.reshape(N).astype(jnp.int32))


# speedup vs baseline: 1.3820x; 1.3820x over previous
import functools

import jax
import jax.numpy as jnp
from jax import lax
from jax.experimental import pallas as pl
from jax.experimental.pallas import tpu as pltpu
from jax.experimental.pallas import tpu_sc as plsc

B, L, V, D = 16384, 200, 128, 8
N = B * L
NC, NS = 2, 16
NW = NC * NS
PER_W = N // NW
CT = 400
SUP = 3200
NSUP = PER_W // SUP
SUBC = SUP // CT
NCHUNK = PER_W // CT
GRP = CT // 16


def _fuse_body(wte_ref, w_ref, b_ref, out_ref):
    out_ref[...] = lax.dot_general(
        wte_ref[...], w_ref[...],
        dimension_numbers=(((1,), (1,)), ((), ())),
        preferred_element_type=jnp.float32,
    ) + b_ref[...]


_fuse_table = pl.pallas_call(
    _fuse_body,
    out_shape=jax.ShapeDtypeStruct((V, D), jnp.float32),
)


def _sc_body(ft_hbm, idx_hbm, out_hbm, tbl_v, idx_v, out_v, sem_in, sem_out):
    wid = lax.axis_index("s") * NC + lax.axis_index("c")
    base = wid * PER_W

    pltpu.sync_copy(ft_hbm, tbl_v)

    pltpu.async_copy(idx_hbm.at[pl.ds(base, SUP)], idx_v.at[pl.ds(0, SUP)],
                     sem_in)

    iota16 = lax.iota(jnp.int32, 16)
    dvecs = [jnp.full((16,), d, jnp.int32) for d in range(D)]

    @pl.loop(0, NSUP)
    def _sup(s):
        isl = (s % 2) * SUP

        pltpu.make_async_copy(
            idx_hbm.at[pl.ds(base + s * SUP, SUP)],
            idx_v.at[pl.ds(isl, SUP)], sem_in).wait()

        @pl.when(s + 1 < NSUP)
        def _():
            pltpu.async_copy(
                idx_hbm.at[pl.ds(base + (s + 1) * SUP, SUP)],
                idx_v.at[pl.ds(SUP - isl, SUP)], sem_in)

        @pl.loop(0, SUBC)
        def _sub(sc):
            c = s * SUBC + sc
            slot = c % 2

            @pl.when(c >= 2)
            def _():
                pltpu.make_async_copy(
                    out_v.at[slot],
                    out_hbm.at[pl.ds(base + (c - 2) * CT, CT), :],
                    sem_out).wait()

            blk = out_v.at[slot]

            @plsc.parallel_loop(0, GRP, unroll=5)
            def _grp(g):
                xv = idx_v[pl.ds(isl + sc * CT + g * 16, 16)]
                gbase = (xv & (V - 1)) * D
                rvec = iota16 + g * 16
                for d in range(D):
                    vals = plsc.load_gather(tbl_v, [gbase + d])
                    plsc.store_scatter(blk, [rvec, dvecs[d]], vals)

            pltpu.async_copy(
                out_v.at[slot],
                out_hbm.at[pl.ds(base + c * CT, CT), :], sem_out)

    for t in (NCHUNK - 2, NCHUNK - 1):
        pltpu.make_async_copy(
            out_v.at[t % 2],
            out_hbm.at[pl.ds(base + t * CT, CT), :], sem_out).wait()


_sc_gather = pl.kernel(
    _sc_body,
    out_type=jax.ShapeDtypeStruct((N, D), jnp.float32),
    mesh=plsc.VectorSubcoreMesh(
        core_axis_name="c", subcore_axis_name="s",
        num_cores=NC, num_subcores=NS),
    compiler_params=pltpu.CompilerParams(needs_layout_passes=False),
    scratch_types=[
        pltpu.VMEM((V * D,), jnp.float32),
        pltpu.VMEM((2 * SUP,), jnp.int32),
        pltpu.VMEM((2, CT, D), jnp.float32),
        pltpu.SemaphoreType.DMA,
        pltpu.SemaphoreType.DMA,
    ],
)


@jax.jit
def kernel(x, wte, W, b):
    ft = _fuse_table(wte, W, b.reshape(1, D))
    out = _sc_gather(ft.reshape(V * D), x.reshape(N).astype(jnp.int32))
    return out.reshape(B, L, D)
